# Initial kernel scaffold; baseline (speedup 1.0000x reference)
import jax
import jax.numpy as jnp
from jax.experimental import pallas as pl


def kernel(x, mask, embedding_matrix):
    raise NotImplementedError("write your pallas kernel here")


# speedup vs baseline: 1.4779x; 1.4779x over previous
import jax
import jax.numpy as jnp
from jax import lax
from jax.experimental import pallas as pl
from jax.experimental.pallas import tpu as pltpu
from jax.experimental.pallas import tpu_sc as plsc

BATCH = 4096
SEQ = 200
EMBED_DIM = 32

NC = 2
NS = 16
NW = NC * NS

TOTAL = BATCH * SEQ
BPW = TOTAL // NW
S = 128
NSTR = BPW // S
G = 8
GROUPS = NSTR // G
GR = G * S


def _gather_body(x_hbm, emb_hbm, out_hbm, idx_v, buf, sem):
    c = lax.axis_index("c")
    s = lax.axis_index("s")
    wid = s * NC + c
    base = wid * BPW

    pltpu.sync_copy(x_hbm.at[wid], idx_v)

    def group(g, carry):
        descs = []
        for j in range(G):
            d = pltpu.async_copy(
                emb_hbm.at[idx_v.at[g * G + j]],
                buf.at[pl.ds(j * S, S)],
                sem,
            )
            descs.append(d)
        for d in descs:
            d.wait()
        pltpu.sync_copy(buf, out_hbm.at[pl.ds(base + g * GR, GR)])
        return carry

    lax.fori_loop(0, GROUPS, group, 0)


def kernel(x, mask, embedding_matrix):
    del mask
    xr = x.astype(jnp.int32).reshape(NW, NSTR, S)
    run = pl.kernel(
        _gather_body,
        out_type=jax.ShapeDtypeStruct((TOTAL, EMBED_DIM), jnp.float32),
        mesh=plsc.VectorSubcoreMesh(core_axis_name="c", subcore_axis_name="s"),
        compiler_params=pltpu.CompilerParams(use_tc_tiling_on_sc=False),
        scratch_types=[
            pltpu.VMEM((NSTR, S), jnp.int32),
            pltpu.VMEM((GR, EMBED_DIM), jnp.float32),
            pltpu.SemaphoreType.DMA,
        ],
    )
    out = run(xr, embedding_matrix)
    return out.reshape(BATCH, SEQ, EMBED_DIM)


# speedup vs baseline: 1.4963x; 1.0124x over previous
import jax
import jax.numpy as jnp
from jax import lax
from jax.experimental import pallas as pl
from jax.experimental.pallas import tpu as pltpu
from jax.experimental.pallas import tpu_sc as plsc

BATCH = 4096
SEQ = 200
EMBED_DIM = 32

NC = 2
NS = 16
NW = NC * NS

TOTAL = BATCH * SEQ
BPW = TOTAL // NW
S = 128
NSTR = BPW // S
G = 5
GROUPS = NSTR // G
GR = G * S
R = 4
Q = GROUPS // R


def _gather_body(x_hbm, emb_hbm, out_hbm, idx_v, bufs, gsems, osems):
    c = lax.axis_index("c")
    s = lax.axis_index("s")
    wid = s * NC + c
    base = wid * BPW

    pltpu.sync_copy(x_hbm.at[wid], idx_v)

    def issue_gathers(g, b):
        for j in range(G):
            pltpu.async_copy(
                emb_hbm.at[idx_v.at[g * G + j]],
                bufs[b].at[pl.ds(j * S, S)],
                gsems[b],
            )

    def wait_gathers(b):
        for j in range(G):
            pltpu.make_async_copy(
                emb_hbm.at[idx_v.at[j]],
                bufs[b].at[pl.ds(j * S, S)],
                gsems[b],
            ).wait()

    def issue_outcopy(g, b):
        pltpu.async_copy(bufs[b], out_hbm.at[pl.ds(base + g * GR, GR)], osems[b])

    def wait_outcopy(b):
        pltpu.make_async_copy(
            bufs[b], out_hbm.at[pl.ds(base, GR)], osems[b]
        ).wait()

    for b in range(R):
        issue_gathers(b, b)

    def ring_round(q, carry):
        for b in range(R):
            wait_gathers(b)
            issue_outcopy(q * R + b, b)
        for b in range(R):
            wait_outcopy(b)
            issue_gathers((q + 1) * R + b, b)
        return carry

    lax.fori_loop(0, Q - 1, ring_round, 0)

    for b in range(R):
        wait_gathers(b)
        issue_outcopy((Q - 1) * R + b, b)
    for b in range(R):
        wait_outcopy(b)


def kernel(x, mask, embedding_matrix):
    del mask
    xr = x.astype(jnp.int32).reshape(NW, NSTR, S)
    run = pl.kernel(
        _gather_body,
        out_type=jax.ShapeDtypeStruct((TOTAL, EMBED_DIM), jnp.float32),
        mesh=plsc.VectorSubcoreMesh(core_axis_name="c", subcore_axis_name="s"),
        compiler_params=pltpu.CompilerParams(use_tc_tiling_on_sc=False),
        scratch_types=[
            pltpu.VMEM((NSTR, S), jnp.int32),
            [pltpu.VMEM((GR, EMBED_DIM), jnp.float32) for _ in range(R)],
            [pltpu.SemaphoreType.DMA for _ in range(R)],
            [pltpu.SemaphoreType.DMA for _ in range(R)],
        ],
    )
    out = run(xr, embedding_matrix)
    return out.reshape(BATCH, SEQ, EMBED_DIM)
